# Initial kernel scaffold; baseline (speedup 1.0000x reference)
import jax
import jax.numpy as jnp
from jax.experimental import pallas as pl


def kernel(bins, weight):
    raise NotImplementedError("write your pallas kernel here")


# speedup vs baseline: 15.5517x; 15.5517x over previous
import jax
import jax.numpy as jnp
from jax import lax
from jax.experimental import pallas as pl
from jax.experimental.pallas import tpu as pltpu
from jax.experimental.pallas import tpu_sc as plsc

N = 2048
H = 16
NUM_BINS = 68
TOTAL = N * N
NC, NS, L = 2, 16, 16
NW = NC * NS
PER_W = TOTAL // NW
CHUNK = 4096
N_CHUNKS = PER_W // CHUNK
TBL = H * NUM_BINS


def _sc_body(weight_hbm, bins_hbm, out_hbm, table_v, bins_v, out_v):
    wid = lax.axis_index("s") * NC + lax.axis_index("c")
    pltpu.sync_copy(weight_hbm, table_v)
    base0 = wid * PER_W

    @pl.loop(0, N_CHUNKS)
    def _chunk(ci):
        base = base0 + ci * CHUNK
        pltpu.sync_copy(bins_hbm.at[pl.ds(base, CHUNK)], bins_v)

        @pl.loop(0, CHUNK // L)
        def _vec(i):
            idx = bins_v[pl.ds(i * L, L)]
            for h in range(H):
                val = plsc.load_gather(table_v, [idx + h * NUM_BINS])
                out_v[pl.ds(h * CHUNK + i * L, L)] = val

        for h in range(H):
            pltpu.sync_copy(out_v.at[pl.ds(h * CHUNK, CHUNK)],
                            out_hbm.at[pl.ds(h * TOTAL + base, CHUNK)])


def kernel(bins, weight):
    k = pl.kernel(
        _sc_body,
        out_type=jax.ShapeDtypeStruct((H * TOTAL,), jnp.float32),
        mesh=plsc.VectorSubcoreMesh(core_axis_name="c", subcore_axis_name="s"),
        compiler_params=pltpu.CompilerParams(needs_layout_passes=False),
        scratch_types=[
            pltpu.VMEM((TBL,), jnp.float32),
            pltpu.VMEM((CHUNK,), jnp.int32),
            pltpu.VMEM((H * CHUNK,), jnp.float32),
        ],
    )
    out = k(weight.reshape(TBL), bins.reshape(TOTAL))
    return out.reshape(H, N, N)


# speedup vs baseline: 35.8543x; 2.3055x over previous
import jax
import jax.numpy as jnp
from jax import lax
from jax.experimental import pallas as pl
from jax.experimental.pallas import tpu as pltpu
from jax.experimental.pallas import tpu_sc as plsc

N = 2048
H = 16
NUM_BINS = 68
TOTAL = N * N
NC, NS, L = 2, 16, 16
NW = NC * NS
PER_W = TOTAL // NW
CHUNK = 2048
N_CHUNKS = PER_W // CHUNK
TBL = H * NUM_BINS


def _sc_body(weight_hbm, bins_hbm, out_hbm, table_v, bins0, bins1,
             out0, out1, si0, si1, so0, so1):
    wid = lax.axis_index("s") * NC + lax.axis_index("c")
    pltpu.sync_copy(weight_hbm, table_v)
    base0 = wid * PER_W
    binsb = (bins0, bins1)
    outb = (out0, out1)
    sin = (si0, si1)
    sout = (so0, so1)

    for b in range(2):
        pltpu.async_copy(bins_hbm.at[pl.ds(base0 + b * CHUNK, CHUNK)],
                         binsb[b], sin[b])

    @pl.loop(0, N_CHUNKS, step=2)
    def _chunk(ci):
        for b in range(2):
            c = ci + b
            base = base0 + c * CHUNK
            bv = binsb[b]
            ov = outb[b]
            pltpu.make_async_copy(bins_hbm.at[pl.ds(0, CHUNK)], bv,
                                  sin[b]).wait()
            @pl.when(c >= 2)
            def _drain():
                for _ in range(H):
                    pltpu.make_async_copy(
                        out_hbm.at[pl.ds(0, CHUNK)],
                        ov.at[pl.ds(0, CHUNK)], sout[b]).wait()

            @plsc.parallel_loop(0, CHUNK // L, unroll=4)
            def _vec(i):
                idx = bv[pl.ds(i * L, L)]
                for h in range(H):
                    ov[pl.ds(h * CHUNK + i * L, L)] = plsc.load_gather(
                        table_v, [idx + h * NUM_BINS])

            @pl.when(c + 2 < N_CHUNKS)
            def _prefetch():
                pltpu.async_copy(
                    bins_hbm.at[pl.ds(base + 2 * CHUNK, CHUNK)], bv, sin[b])

            for h in range(H):
                pltpu.async_copy(ov.at[pl.ds(h * CHUNK, CHUNK)],
                                 out_hbm.at[pl.ds(h * TOTAL + base, CHUNK)],
                                 sout[b])

    for b in range(2):
        for _ in range(H):
            pltpu.make_async_copy(out_hbm.at[pl.ds(0, CHUNK)],
                                  outb[b].at[pl.ds(0, CHUNK)],
                                  sout[b]).wait()


def kernel(bins, weight):
    k = pl.kernel(
        _sc_body,
        out_type=jax.ShapeDtypeStruct((H * TOTAL,), jnp.float32),
        mesh=plsc.VectorSubcoreMesh(core_axis_name="c", subcore_axis_name="s"),
        compiler_params=pltpu.CompilerParams(needs_layout_passes=False),
        scratch_types=[
            pltpu.VMEM((TBL,), jnp.float32),
            pltpu.VMEM((CHUNK,), jnp.int32),
            pltpu.VMEM((CHUNK,), jnp.int32),
            pltpu.VMEM((H * CHUNK,), jnp.float32),
            pltpu.VMEM((H * CHUNK,), jnp.float32),
            pltpu.SemaphoreType.DMA,
            pltpu.SemaphoreType.DMA,
            pltpu.SemaphoreType.DMA,
            pltpu.SemaphoreType.DMA,
        ],
    )
    out = k(weight.reshape(TBL), bins.reshape(TOTAL))
    return out.reshape(H, N, N)


# speedup vs baseline: 119.9695x; 3.3460x over previous
import jax
import jax.numpy as jnp
from jax import lax
from jax.experimental import pallas as pl
from jax.experimental.pallas import tpu as pltpu
from jax.experimental.pallas import tpu_sc as plsc

N = 2048
H = 16
NUM_BINS = 68
NC, NS, L = 2, 16, 16
NW = NC * NS
ROWS_PER_W = N // NW
TBL = H * NUM_BINS


def _sc_body(weight_hbm, bins_hbm, out_hbm, table_v, bins0, bins1,
             out0, out1, si0, si1, so0, so1):
    wid = lax.axis_index("s") * NC + lax.axis_index("c")
    pltpu.sync_copy(weight_hbm, table_v)
    row0 = wid * ROWS_PER_W
    binsb = (bins0, bins1)
    outb = (out0, out1)
    sin = (si0, si1)
    sout = (so0, so1)

    for b in range(2):
        pltpu.async_copy(bins_hbm.at[pl.ds(row0 + b, 1), :], binsb[b], sin[b])

    @pl.loop(0, ROWS_PER_W, step=2)
    def _row(ci):
        for b in range(2):
            c = ci + b
            row = row0 + c
            bv = binsb[b]
            ov = outb[b]
            pltpu.make_async_copy(bins_hbm.at[pl.ds(0, 1), :], bv,
                                  sin[b]).wait()
            @pl.when(c >= 2)
            def _drain():
                for _ in range(H):
                    pltpu.make_async_copy(
                        out_hbm.at[pl.ds(0, 1), pl.ds(0, 1), :],
                        ov.at[pl.ds(0, 1)], sout[b]).wait()

            @plsc.parallel_loop(0, N // L, unroll=4)
            def _vec(i):
                idx = bv[0, pl.ds(i * L, L)]
                for h in range(H):
                    ov[h, 0, pl.ds(i * L, L)] = plsc.load_gather(
                        table_v, [idx + h * NUM_BINS])

            @pl.when(c + 2 < ROWS_PER_W)
            def _prefetch():
                pltpu.async_copy(bins_hbm.at[pl.ds(row + 2, 1), :], bv,
                                 sin[b])

            for h in range(H):
                pltpu.async_copy(ov.at[pl.ds(h, 1)],
                                 out_hbm.at[pl.ds(h, 1), pl.ds(row, 1), :],
                                 sout[b])

    for b in range(2):
        for _ in range(H):
            pltpu.make_async_copy(out_hbm.at[pl.ds(0, 1), pl.ds(0, 1), :],
                                  outb[b].at[pl.ds(0, 1)],
                                  sout[b]).wait()


def kernel(bins, weight):
    k = pl.kernel(
        _sc_body,
        out_type=jax.ShapeDtypeStruct((H, N, N), jnp.float32),
        mesh=plsc.VectorSubcoreMesh(core_axis_name="c", subcore_axis_name="s"),
        compiler_params=pltpu.CompilerParams(needs_layout_passes=False),
        scratch_types=[
            pltpu.VMEM((TBL,), jnp.float32),
            pltpu.VMEM((1, N), jnp.int32),
            pltpu.VMEM((1, N), jnp.int32),
            pltpu.VMEM((H, 1, N), jnp.float32),
            pltpu.VMEM((H, 1, N), jnp.float32),
            pltpu.SemaphoreType.DMA,
            pltpu.SemaphoreType.DMA,
            pltpu.SemaphoreType.DMA,
            pltpu.SemaphoreType.DMA,
        ],
    )
    return k(weight.reshape(TBL), bins)
